# Initial kernel scaffold; baseline (speedup 1.0000x reference)
import jax
import jax.numpy as jnp
from jax.experimental import pallas as pl


def kernel(x, weight):
    raise NotImplementedError("write your pallas kernel here")


# speedup vs baseline: 26.4597x; 26.4597x over previous
import jax
import jax.numpy as jnp
from jax import lax
from jax.experimental import pallas as pl
from jax.experimental.pallas import tpu as pltpu

_B, _L = 1024, 2048
_V, _C = 27, 26
_BB = 8


def _body(x_ref, w_ref, o_ref):
    w = w_ref[...]
    iota = lax.broadcasted_iota(jnp.int32, (_V, _L), 0)
    for i in range(_BB):
        xi = x_ref[i, :][None, :]
        oh = (iota == xi).astype(jnp.float32)
        o_ref[i] = lax.dot_general(
            w, oh, (((0,), (0,)), ((), ())),
            preferred_element_type=jnp.float32)


def kernel(x, weight):
    grid = (_B // _BB,)
    return pl.pallas_call(
        _body,
        grid=grid,
        in_specs=[
            pl.BlockSpec((_BB, _L), lambda i: (i, 0)),
            pl.BlockSpec((_V, _C), lambda i: (0, 0)),
        ],
        out_specs=pl.BlockSpec((_BB, _C, _L), lambda i: (i, 0, 0)),
        out_shape=jax.ShapeDtypeStruct((_B, _C, _L), jnp.float32),
        compiler_params=pltpu.CompilerParams(
            dimension_semantics=("parallel",)),
    )(x, weight)


# speedup vs baseline: 31.3152x; 1.1835x over previous
import jax
import jax.numpy as jnp
from jax import lax
from jax.experimental import pallas as pl
from jax.experimental.pallas import tpu as pltpu

_B, _L = 1024, 2048
_V, _C = 27, 26
_BB = 64


def _body(x_ref, w_ref, o_ref):
    w = w_ref[...]
    iota = lax.broadcasted_iota(jnp.int32, (_V, _L), 0)
    for i in range(_BB):
        xi = x_ref[i, :][None, :]
        oh = (iota == xi).astype(jnp.float32)
        o_ref[i] = lax.dot_general(
            w, oh, (((0,), (0,)), ((), ())),
            preferred_element_type=jnp.float32)


def kernel(x, weight):
    grid = (_B // _BB,)
    return pl.pallas_call(
        _body,
        grid=grid,
        in_specs=[
            pl.BlockSpec((_BB, _L), lambda i: (i, 0)),
            pl.BlockSpec((_V, _C), lambda i: (0, 0)),
        ],
        out_specs=pl.BlockSpec((_BB, _C, _L), lambda i: (i, 0, 0)),
        out_shape=jax.ShapeDtypeStruct((_B, _C, _L), jnp.float32),
        compiler_params=pltpu.CompilerParams(
            dimension_semantics=("parallel",)),
    )(x, weight)
